# baseline (speedup 1.0000x reference)
import functools

import jax
import jax.numpy as jnp
from jax import lax
from jax.experimental import pallas as pl
from jax.experimental.pallas import tpu as pltpu
from jax.experimental.pallas import tpu_sc as plsc

_N, _D = 2048, 768
_E, _K, _F = 8, 2, 2048

_BT = 128
_R = _K * _N + _E * _BT
_NB = _R // _BT
_CH = _K * _N // 32
_FC = 1024

_NEG_INF = float("-inf")


def _chunked_cumsum(ob, t128, t16):
    parts = []
    tots = []
    for c in range(16):
        blk = ob[c * 128:(c + 1) * 128, :]
        cc = jnp.dot(t128, blk, preferred_element_type=jnp.float32)
        parts.append(cc)
        tots.append(cc[127:128, :])
    s = jnp.concatenate(tots, axis=0)
    bases = jnp.dot(t16, s.astype(jnp.bfloat16),
                    preferred_element_type=jnp.float32)
    out = jnp.concatenate(
        [parts[c] + bases[c:c + 1, :] for c in range(16)], axis=0)
    total = bases[15:16, :] + s[15:16, :]
    return out, total


def _routing_body(logits_ref, p0_ref, p1_ref, w0_ref, w1_ref, blk_ref):
    lg = logits_ref[...]
    lanes = lax.broadcasted_iota(jnp.int32, (_N, _E), 1)
    m1 = jnp.max(lg, axis=1, keepdims=True)
    i1 = jnp.min(jnp.where(lg == m1, lanes, _E), axis=1, keepdims=True)
    masked = jnp.where(lanes == i1, _NEG_INF, lg)
    m2 = jnp.max(masked, axis=1, keepdims=True)
    i2 = jnp.min(jnp.where(masked == m2, lanes, _E), axis=1, keepdims=True)

    r = jnp.exp(m2 - m1)
    w0_ref[...] = 1.0 / (1.0 + r)
    w1_ref[...] = r / (1.0 + r)

    o0 = (lanes == i1).astype(jnp.bfloat16)
    o1 = (lanes == i2).astype(jnp.bfloat16)

    ri = lax.broadcasted_iota(jnp.int32, (128, 128), 0)
    ci = lax.broadcasted_iota(jnp.int32, (128, 128), 1)
    t128 = (ci <= ri).astype(jnp.bfloat16)
    ri16 = lax.broadcasted_iota(jnp.int32, (16, 16), 0)
    ci16 = lax.broadcasted_iota(jnp.int32, (16, 16), 1)
    t16 = (ri16 > ci16).astype(jnp.bfloat16)
    c0, cnt0 = _chunked_cumsum(o0, t128, t16)
    c1, cnt1 = _chunked_cumsum(o1, t128, t16)

    cnt = (cnt0 + cnt1).astype(jnp.int32)
    pc = ((cnt + (_BT - 1)) // _BT) * _BT
    pcb = jnp.broadcast_to(pc.astype(jnp.bfloat16), (_E, _E))
    ri8 = lax.broadcasted_iota(jnp.int32, (_E, _E), 0)
    ci8 = lax.broadcasted_iota(jnp.int32, (_E, _E), 1)
    t8 = (ri8 < ci8).astype(jnp.bfloat16)
    pb = jnp.dot(pcb, t8, preferred_element_type=jnp.float32)[0:1, :]

    p0 = jnp.sum(o0.astype(jnp.float32) * (pb + c0), axis=1, keepdims=True) - 1.0
    p1 = jnp.sum(o1.astype(jnp.float32) * (pb + cnt0 + c1),
                 axis=1, keepdims=True) - 1.0
    p0_ref[...] = p0.astype(jnp.int32)
    p1_ref[...] = p1.astype(jnp.int32)

    jrow = lax.broadcasted_iota(jnp.int32, (128, _E), 0) * _BT
    jrowf = jrow.astype(jnp.float32)
    nonempty = pc > 0
    be = jnp.sum((pb <= jrowf).astype(jnp.int32), axis=1, keepdims=True) - 1
    started = jnp.logical_and(pb <= jrowf, nonempty)
    rix = jnp.sum(started.astype(jnp.int32), axis=1, keepdims=True) - 1
    lanes8 = lax.broadcasted_iota(jnp.int32, (128, _E), 1)
    nxt_cand = jnp.where(jnp.logical_and(pb > jrowf, nonempty), lanes8, 99)
    nxt = jnp.min(nxt_cand, axis=1, keepdims=True)
    nxt = jnp.where(nxt == 99, -1, nxt)
    total = jnp.sum(pc, axis=1, keepdims=True)
    nbu = jnp.broadcast_to(total // _BT, (128, 1))
    lane_id = lax.broadcasted_iota(jnp.int32, (128, 4), 1)
    blk_ref[...] = jnp.where(
        lane_id == 0, be,
        jnp.where(lane_id == 1, nxt, jnp.where(lane_id == 2, rix, nbu)))


def _routing(logits):
    return pl.pallas_call(
        _routing_body,
        out_shape=(
            jax.ShapeDtypeStruct((_N, 1), jnp.int32),
            jax.ShapeDtypeStruct((_N, 1), jnp.int32),
            jax.ShapeDtypeStruct((_N, 1), jnp.float32),
            jax.ShapeDtypeStruct((_N, 1), jnp.float32),
            jax.ShapeDtypeStruct((128, 4), jnp.int32),
        ),
    )(logits)


def _sc_dispatch(xf, ppos):
    mesh = plsc.VectorSubcoreMesh(core_axis_name="c", subcore_axis_name="s")

    @functools.partial(
        pl.kernel,
        out_type=jax.ShapeDtypeStruct((_R, _D), jnp.float32),
        mesh=mesh,
        scratch_types=[
            pltpu.VMEM((_CH,), jnp.int32),
            pltpu.VMEM((_CH, _D), jnp.float32),
            pltpu.SemaphoreType.DMA,
        ],
    )
    def k(x_hbm, idx_hbm, out_hbm, idx_v, rows_v, sem):
        wid = lax.axis_index("s") * 2 + lax.axis_index("c")
        base = wid * _CH
        tok = lax.rem(base, _N)
        pltpu.sync_copy(idx_hbm.at[pl.ds(base, _CH)], idx_v)
        pltpu.async_copy(x_hbm.at[pl.ds(tok, _CH)], rows_v, sem).wait()
        pltpu.sync_copy(rows_v, out_hbm.at[idx_v])

    return k(xf, ppos)


def _sc_collect(y, ppos):
    mesh = plsc.VectorSubcoreMesh(core_axis_name="c", subcore_axis_name="s")

    @functools.partial(
        pl.kernel,
        out_type=jax.ShapeDtypeStruct((_K * _N, _D), jnp.float32),
        mesh=mesh,
        scratch_types=[
            pltpu.VMEM((_CH,), jnp.int32),
            pltpu.VMEM((_CH, _D), jnp.float32),
            pltpu.SemaphoreType.DMA,
        ],
    )
    def k(y_hbm, idx_hbm, out_hbm, idx_v, rows_v, sem):
        wid = lax.axis_index("s") * 2 + lax.axis_index("c")
        base = wid * _CH
        pltpu.sync_copy(idx_hbm.at[pl.ds(base, _CH)], idx_v)
        pltpu.async_copy(y_hbm.at[idx_v], rows_v, sem).wait()
        pltpu.sync_copy(rows_v, out_hbm.at[pl.ds(base, _CH)])

    return k(y, ppos)


def _start_weights(w1_ref, w3_ref, w2_ref, w1f, w3f, w2f, sems, e, slot):
    pltpu.make_async_copy(w1_ref.at[e], w1f.at[slot], sems.at[slot]).start()
    pltpu.make_async_copy(w3_ref.at[e], w3f.at[slot], sems.at[slot]).start()
    pltpu.make_async_copy(w2_ref.at[e], w2f.at[slot], sems.at[slot]).start()


def _wait_weights(w1_ref, w3_ref, w2_ref, w1f, w3f, w2f, sems, e, slot):
    pltpu.make_async_copy(w1_ref.at[e], w1f.at[slot], sems.at[slot]).wait()
    pltpu.make_async_copy(w3_ref.at[e], w3f.at[slot], sems.at[slot]).wait()
    pltpu.make_async_copy(w2_ref.at[e], w2f.at[slot], sems.at[slot]).wait()


def _ffn_body(be_ref, nxt_ref, rix_ref, nbu_ref, xg_ref,
              w1_ref, w3_ref, w2_ref, y_ref,
              w1f, w3f, w2f, w1s_ref, w3s_ref, w2s_ref, sems):
    j = pl.program_id(0)

    @pl.when(j < nbu_ref[0])
    def _():
        cur = be_ref[j]
        slot = lax.rem(rix_ref[j], 2)
        runfirst = jnp.logical_or(j == 0, be_ref[jnp.maximum(j - 1, 0)] != cur)

        @pl.when(j == 0)
        def _():
            _start_weights(w1_ref, w3_ref, w2_ref, w1f, w3f, w2f, sems,
                           cur, slot)

        @pl.when(jnp.logical_and(runfirst, nxt_ref[j] >= 0))
        def _():
            _start_weights(w1_ref, w3_ref, w2_ref, w1f, w3f, w2f, sems,
                           nxt_ref[j], 1 - slot)

        @pl.when(runfirst)
        def _():
            _wait_weights(w1_ref, w3_ref, w2_ref, w1f, w3f, w2f, sems,
                          cur, slot)
            w1s_ref[...] = w1f[slot].astype(jnp.bfloat16)
            w3s_ref[...] = w3f[slot].astype(jnp.bfloat16)
            w2s_ref[...] = w2f[slot].astype(jnp.bfloat16)

        xb = xg_ref[...].astype(jnp.bfloat16)
        acc = jnp.zeros((_BT, _D), jnp.float32)
        for f in range(_F // _FC):
            fs = pl.ds(f * _FC, _FC)
            a = jnp.dot(xb, w1s_ref[:, fs], preferred_element_type=jnp.float32)
            b = jnp.dot(xb, w3s_ref[:, fs], preferred_element_type=jnp.float32)
            h = (jax.nn.silu(a) * b).astype(jnp.bfloat16)
            acc = acc + jnp.dot(h, w2s_ref[fs, :],
                                preferred_element_type=jnp.float32)
        y_ref[...] = acc


def _ffn(blkexp, nxtrun, runix, nbused, xg, W1, W3, W2):
    return pl.pallas_call(
        _ffn_body,
        grid_spec=pltpu.PrefetchScalarGridSpec(
            num_scalar_prefetch=4,
            grid=(_NB,),
            in_specs=[
                pl.BlockSpec((_BT, _D), lambda j, *_: (j, 0)),
                pl.BlockSpec(memory_space=pltpu.MemorySpace.HBM),
                pl.BlockSpec(memory_space=pltpu.MemorySpace.HBM),
                pl.BlockSpec(memory_space=pltpu.MemorySpace.HBM),
            ],
            out_specs=pl.BlockSpec((_BT, _D), lambda j, *_: (j, 0)),
            scratch_shapes=[
                pltpu.VMEM((2, _D, _F), jnp.float32),
                pltpu.VMEM((2, _D, _F), jnp.float32),
                pltpu.VMEM((2, _F, _D), jnp.float32),
                pltpu.VMEM((_D, _F), jnp.bfloat16),
                pltpu.VMEM((_D, _F), jnp.bfloat16),
                pltpu.VMEM((_F, _D), jnp.bfloat16),
                pltpu.SemaphoreType.DMA((2,)),
            ],
        ),
        out_shape=jax.ShapeDtypeStruct((_R, _D), jnp.float32),
        compiler_params=pltpu.CompilerParams(
            dimension_semantics=("arbitrary",),
        ),
    )(blkexp, nxtrun, runix, nbused, xg, W1, W3, W2)


_BC = 256


def _combine_body(y0_ref, y1_ref, w0_ref, w1_ref, o_ref):
    o_ref[...] = w0_ref[...] * y0_ref[...] + w1_ref[...] * y1_ref[...]


def _combine(yg, w0, w1):
    nb = _N // _BC
    return pl.pallas_call(
        _combine_body,
        grid=(nb,),
        in_specs=[
            pl.BlockSpec((_BC, _D), lambda t: (t, 0)),
            pl.BlockSpec((_BC, _D), lambda t: (t + nb, 0)),
            pl.BlockSpec((_BC, 1), lambda t: (t, 0)),
            pl.BlockSpec((_BC, 1), lambda t: (t, 0)),
        ],
        out_specs=pl.BlockSpec((_BC, _D), lambda t: (t, 0)),
        out_shape=jax.ShapeDtypeStruct((_N, _D), jnp.float32),
        compiler_params=pltpu.CompilerParams(
            dimension_semantics=("parallel",),
        ),
    )(yg, yg, w0, w1)


def kernel(x, Wg, W1, W3, W2):
    xf = x.reshape(_N, _D)
    logits = xf @ Wg
    p0, p1, w0, w1, blk = _routing(logits)
    ppos = jnp.concatenate([p0, p1], axis=0).reshape(_K * _N)
    blkexp = blk[:_NB, 0]
    nxtrun = blk[:_NB, 1]
    runix = blk[:_NB, 2]
    nbused = blk[0:1, 3].reshape(1)
    xg = _sc_dispatch(xf, ppos)
    y = _ffn(blkexp, nxtrun, runix, nbused, xg, W1, W3, W2)
    yg = _sc_collect(y, ppos)
    out = _combine(yg, w0, w1)
    return out.reshape(x.shape)


# speedup vs baseline: 1.1318x; 1.1318x over previous
import functools

import jax
import jax.numpy as jnp
from jax import lax
from jax.experimental import pallas as pl
from jax.experimental.pallas import tpu as pltpu
from jax.experimental.pallas import tpu_sc as plsc

_N, _D = 2048, 768
_E, _K, _F = 8, 2, 2048

_BT = 256
_R = _K * _N + _E * _BT
_NB = _R // _BT
_CH = _K * _N // 32
_FC = 1024

_NEG_INF = float("-inf")


def _chunked_cumsum(ob, t128, t16):
    parts = []
    tots = []
    for c in range(16):
        blk = ob[c * 128:(c + 1) * 128, :]
        cc = jnp.dot(t128, blk, preferred_element_type=jnp.float32)
        parts.append(cc)
        tots.append(cc[127:128, :])
    s = jnp.concatenate(tots, axis=0)
    bases = jnp.dot(t16, s.astype(jnp.bfloat16),
                    preferred_element_type=jnp.float32)
    out = jnp.concatenate(
        [parts[c] + bases[c:c + 1, :] for c in range(16)], axis=0)
    total = bases[15:16, :] + s[15:16, :]
    return out, total


def _routing_body(logits_ref, p0_ref, p1_ref, w0_ref, w1_ref, blk_ref):
    lg = logits_ref[...]
    lanes = lax.broadcasted_iota(jnp.int32, (_N, _E), 1)
    m1 = jnp.max(lg, axis=1, keepdims=True)
    i1 = jnp.min(jnp.where(lg == m1, lanes, _E), axis=1, keepdims=True)
    masked = jnp.where(lanes == i1, _NEG_INF, lg)
    m2 = jnp.max(masked, axis=1, keepdims=True)
    i2 = jnp.min(jnp.where(masked == m2, lanes, _E), axis=1, keepdims=True)

    r = jnp.exp(m2 - m1)
    w0_ref[...] = 1.0 / (1.0 + r)
    w1_ref[...] = r / (1.0 + r)

    o0 = (lanes == i1).astype(jnp.bfloat16)
    o1 = (lanes == i2).astype(jnp.bfloat16)

    ri = lax.broadcasted_iota(jnp.int32, (128, 128), 0)
    ci = lax.broadcasted_iota(jnp.int32, (128, 128), 1)
    t128 = (ci <= ri).astype(jnp.bfloat16)
    ri16 = lax.broadcasted_iota(jnp.int32, (16, 16), 0)
    ci16 = lax.broadcasted_iota(jnp.int32, (16, 16), 1)
    t16 = (ri16 > ci16).astype(jnp.bfloat16)
    c0, cnt0 = _chunked_cumsum(o0, t128, t16)
    c1, cnt1 = _chunked_cumsum(o1, t128, t16)

    cnt = (cnt0 + cnt1).astype(jnp.int32)
    pc = ((cnt + (_BT - 1)) // _BT) * _BT
    pcb = jnp.broadcast_to(pc.astype(jnp.bfloat16), (_E, _E))
    ri8 = lax.broadcasted_iota(jnp.int32, (_E, _E), 0)
    ci8 = lax.broadcasted_iota(jnp.int32, (_E, _E), 1)
    t8 = (ri8 < ci8).astype(jnp.bfloat16)
    pb = jnp.dot(pcb, t8, preferred_element_type=jnp.float32)[0:1, :]

    p0 = jnp.sum(o0.astype(jnp.float32) * (pb + c0), axis=1, keepdims=True) - 1.0
    p1 = jnp.sum(o1.astype(jnp.float32) * (pb + cnt0 + c1),
                 axis=1, keepdims=True) - 1.0
    p0_ref[...] = p0.astype(jnp.int32)
    p1_ref[...] = p1.astype(jnp.int32)

    jrow = lax.broadcasted_iota(jnp.int32, (128, _E), 0) * _BT
    jrowf = jrow.astype(jnp.float32)
    nonempty = pc > 0
    be = jnp.sum((pb <= jrowf).astype(jnp.int32), axis=1, keepdims=True) - 1
    started = jnp.logical_and(pb <= jrowf, nonempty)
    rix = jnp.sum(started.astype(jnp.int32), axis=1, keepdims=True) - 1
    lanes8 = lax.broadcasted_iota(jnp.int32, (128, _E), 1)
    nxt_cand = jnp.where(jnp.logical_and(pb > jrowf, nonempty), lanes8, 99)
    nxt = jnp.min(nxt_cand, axis=1, keepdims=True)
    nxt = jnp.where(nxt == 99, -1, nxt)
    total = jnp.sum(pc, axis=1, keepdims=True)
    nbu = jnp.broadcast_to(total // _BT, (128, 1))
    lane_id = lax.broadcasted_iota(jnp.int32, (128, 4), 1)
    blk_ref[...] = jnp.where(
        lane_id == 0, be,
        jnp.where(lane_id == 1, nxt, jnp.where(lane_id == 2, rix, nbu)))


def _routing(logits):
    return pl.pallas_call(
        _routing_body,
        out_shape=(
            jax.ShapeDtypeStruct((_N, 1), jnp.int32),
            jax.ShapeDtypeStruct((_N, 1), jnp.int32),
            jax.ShapeDtypeStruct((_N, 1), jnp.float32),
            jax.ShapeDtypeStruct((_N, 1), jnp.float32),
            jax.ShapeDtypeStruct((128, 4), jnp.int32),
        ),
    )(logits)


def _sc_dispatch(xf, ppos):
    mesh = plsc.VectorSubcoreMesh(core_axis_name="c", subcore_axis_name="s")

    @functools.partial(
        pl.kernel,
        out_type=jax.ShapeDtypeStruct((_R, _D), jnp.float32),
        mesh=mesh,
        scratch_types=[
            pltpu.VMEM((_CH,), jnp.int32),
            pltpu.VMEM((_CH, _D), jnp.float32),
            pltpu.SemaphoreType.DMA,
        ],
    )
    def k(x_hbm, idx_hbm, out_hbm, idx_v, rows_v, sem):
        wid = lax.axis_index("s") * 2 + lax.axis_index("c")
        base = wid * _CH
        tok = lax.rem(base, _N)
        pltpu.sync_copy(idx_hbm.at[pl.ds(base, _CH)], idx_v)
        pltpu.async_copy(x_hbm.at[pl.ds(tok, _CH)], rows_v, sem).wait()
        pltpu.sync_copy(rows_v, out_hbm.at[idx_v])

    return k(xf, ppos)


def _sc_collect(y, ppos):
    mesh = plsc.VectorSubcoreMesh(core_axis_name="c", subcore_axis_name="s")

    @functools.partial(
        pl.kernel,
        out_type=jax.ShapeDtypeStruct((_K * _N, _D), jnp.float32),
        mesh=mesh,
        scratch_types=[
            pltpu.VMEM((_CH,), jnp.int32),
            pltpu.VMEM((_CH, _D), jnp.float32),
            pltpu.SemaphoreType.DMA,
        ],
    )
    def k(y_hbm, idx_hbm, out_hbm, idx_v, rows_v, sem):
        wid = lax.axis_index("s") * 2 + lax.axis_index("c")
        base = wid * _CH
        pltpu.sync_copy(idx_hbm.at[pl.ds(base, _CH)], idx_v)
        pltpu.async_copy(y_hbm.at[idx_v], rows_v, sem).wait()
        pltpu.sync_copy(rows_v, out_hbm.at[pl.ds(base, _CH)])

    return k(y, ppos)


def _start_weights(w1_ref, w3_ref, w2_ref, w1f, w3f, w2f, sems, e, slot):
    pltpu.make_async_copy(w1_ref.at[e], w1f.at[slot], sems.at[slot]).start()
    pltpu.make_async_copy(w3_ref.at[e], w3f.at[slot], sems.at[slot]).start()
    pltpu.make_async_copy(w2_ref.at[e], w2f.at[slot], sems.at[slot]).start()


def _wait_weights(w1_ref, w3_ref, w2_ref, w1f, w3f, w2f, sems, e, slot):
    pltpu.make_async_copy(w1_ref.at[e], w1f.at[slot], sems.at[slot]).wait()
    pltpu.make_async_copy(w3_ref.at[e], w3f.at[slot], sems.at[slot]).wait()
    pltpu.make_async_copy(w2_ref.at[e], w2f.at[slot], sems.at[slot]).wait()


def _ffn_body(be_ref, nxt_ref, rix_ref, nbu_ref, xg_ref,
              w1_ref, w3_ref, w2_ref, y_ref,
              w1f, w3f, w2f, sems):
    j = pl.program_id(0)

    @pl.when(j < nbu_ref[0])
    def _():
        cur = be_ref[j]
        slot = lax.rem(rix_ref[j], 2)
        runfirst = jnp.logical_or(j == 0, be_ref[jnp.maximum(j - 1, 0)] != cur)

        @pl.when(j == 0)
        def _():
            _start_weights(w1_ref, w3_ref, w2_ref, w1f, w3f, w2f, sems,
                           cur, slot)

        @pl.when(jnp.logical_and(runfirst, nxt_ref[j] >= 0))
        def _():
            _start_weights(w1_ref, w3_ref, w2_ref, w1f, w3f, w2f, sems,
                           nxt_ref[j], 1 - slot)

        @pl.when(runfirst)
        def _():
            _wait_weights(w1_ref, w3_ref, w2_ref, w1f, w3f, w2f, sems,
                          cur, slot)

        xb = xg_ref[...]
        acc = jnp.zeros((_BT, _D), jnp.float32)
        for f in range(_F // _FC):
            fs = pl.ds(f * _FC, _FC)
            a = jnp.dot(xb, w1f[slot, :, fs],
                        preferred_element_type=jnp.float32)
            b = jnp.dot(xb, w3f[slot, :, fs],
                        preferred_element_type=jnp.float32)
            h = jax.nn.silu(a) * b
            acc = acc + jnp.dot(h, w2f[slot, fs, :],
                                preferred_element_type=jnp.float32)
        y_ref[...] = acc


def _ffn(blkexp, nxtrun, runix, nbused, xg, W1, W3, W2):
    return pl.pallas_call(
        _ffn_body,
        grid_spec=pltpu.PrefetchScalarGridSpec(
            num_scalar_prefetch=4,
            grid=(_NB,),
            in_specs=[
                pl.BlockSpec((_BT, _D), lambda j, *_: (j, 0)),
                pl.BlockSpec(memory_space=pltpu.MemorySpace.HBM),
                pl.BlockSpec(memory_space=pltpu.MemorySpace.HBM),
                pl.BlockSpec(memory_space=pltpu.MemorySpace.HBM),
            ],
            out_specs=pl.BlockSpec((_BT, _D), lambda j, *_: (j, 0)),
            scratch_shapes=[
                pltpu.VMEM((2, _D, _F), jnp.float32),
                pltpu.VMEM((2, _D, _F), jnp.float32),
                pltpu.VMEM((2, _F, _D), jnp.float32),
                pltpu.SemaphoreType.DMA((2,)),
            ],
        ),
        out_shape=jax.ShapeDtypeStruct((_R, _D), jnp.float32),
        compiler_params=pltpu.CompilerParams(
            dimension_semantics=("arbitrary",),
        ),
    )(blkexp, nxtrun, runix, nbused, xg, W1, W3, W2)


_BC = 256


def _combine_body(y0_ref, y1_ref, w0_ref, w1_ref, o_ref):
    o_ref[...] = w0_ref[...] * y0_ref[...] + w1_ref[...] * y1_ref[...]


def _combine(yg, w0, w1):
    nb = _N // _BC
    return pl.pallas_call(
        _combine_body,
        grid=(nb,),
        in_specs=[
            pl.BlockSpec((_BC, _D), lambda t: (t, 0)),
            pl.BlockSpec((_BC, _D), lambda t: (t + nb, 0)),
            pl.BlockSpec((_BC, 1), lambda t: (t, 0)),
            pl.BlockSpec((_BC, 1), lambda t: (t, 0)),
        ],
        out_specs=pl.BlockSpec((_BC, _D), lambda t: (t, 0)),
        out_shape=jax.ShapeDtypeStruct((_N, _D), jnp.float32),
        compiler_params=pltpu.CompilerParams(
            dimension_semantics=("parallel",),
        ),
    )(yg, yg, w0, w1)


def kernel(x, Wg, W1, W3, W2):
    xf = x.reshape(_N, _D)
    logits = xf @ Wg
    p0, p1, w0, w1, blk = _routing(logits)
    ppos = jnp.concatenate([p0, p1], axis=0).reshape(_K * _N)
    blkexp = blk[:_NB, 0]
    nxtrun = blk[:_NB, 1]
    runix = blk[:_NB, 2]
    nbused = blk[0:1, 3].reshape(1)
    xg = _sc_dispatch(xf, ppos)
    y = _ffn(blkexp, nxtrun, runix, nbused, xg, W1, W3, W2)
    yg = _sc_collect(y, ppos)
    out = _combine(yg, w0, w1)
    return out.reshape(x.shape)


# speedup vs baseline: 1.1445x; 1.0112x over previous
import functools

import jax
import jax.numpy as jnp
from jax import lax
from jax.experimental import pallas as pl
from jax.experimental.pallas import tpu as pltpu
from jax.experimental.pallas import tpu_sc as plsc

_N, _D = 2048, 768
_E, _K, _F = 8, 2, 2048

_BT = 256
_R = _K * _N + _E * _BT
_NB = _R // _BT
_CH = _K * _N // 32
_FC = 512

_NEG_INF = float("-inf")


def _chunked_cumsum(ob, t128, t16):
    parts = []
    tots = []
    for c in range(16):
        blk = ob[c * 128:(c + 1) * 128, :]
        cc = jnp.dot(t128, blk, preferred_element_type=jnp.float32)
        parts.append(cc)
        tots.append(cc[127:128, :])
    s = jnp.concatenate(tots, axis=0)
    bases = jnp.dot(t16, s.astype(jnp.bfloat16),
                    preferred_element_type=jnp.float32)
    out = jnp.concatenate(
        [parts[c] + bases[c:c + 1, :] for c in range(16)], axis=0)
    total = bases[15:16, :] + s[15:16, :]
    return out, total


def _routing_body(logits_ref, p0_ref, p1_ref, w0_ref, w1_ref, blk_ref):
    lg = logits_ref[...]
    lanes = lax.broadcasted_iota(jnp.int32, (_N, _E), 1)
    m1 = jnp.max(lg, axis=1, keepdims=True)
    i1 = jnp.min(jnp.where(lg == m1, lanes, _E), axis=1, keepdims=True)
    masked = jnp.where(lanes == i1, _NEG_INF, lg)
    m2 = jnp.max(masked, axis=1, keepdims=True)
    i2 = jnp.min(jnp.where(masked == m2, lanes, _E), axis=1, keepdims=True)

    r = jnp.exp(m2 - m1)
    w0_ref[...] = 1.0 / (1.0 + r)
    w1_ref[...] = r / (1.0 + r)

    o0 = (lanes == i1).astype(jnp.bfloat16)
    o1 = (lanes == i2).astype(jnp.bfloat16)

    ri = lax.broadcasted_iota(jnp.int32, (128, 128), 0)
    ci = lax.broadcasted_iota(jnp.int32, (128, 128), 1)
    t128 = (ci <= ri).astype(jnp.bfloat16)
    ri16 = lax.broadcasted_iota(jnp.int32, (16, 16), 0)
    ci16 = lax.broadcasted_iota(jnp.int32, (16, 16), 1)
    t16 = (ri16 > ci16).astype(jnp.bfloat16)
    c0, cnt0 = _chunked_cumsum(o0, t128, t16)
    c1, cnt1 = _chunked_cumsum(o1, t128, t16)

    cnt = (cnt0 + cnt1).astype(jnp.int32)
    pc = ((cnt + (_BT - 1)) // _BT) * _BT
    pcb = jnp.broadcast_to(pc.astype(jnp.bfloat16), (_E, _E))
    ri8 = lax.broadcasted_iota(jnp.int32, (_E, _E), 0)
    ci8 = lax.broadcasted_iota(jnp.int32, (_E, _E), 1)
    t8 = (ri8 < ci8).astype(jnp.bfloat16)
    pb = jnp.dot(pcb, t8, preferred_element_type=jnp.float32)[0:1, :]

    p0 = jnp.sum(o0.astype(jnp.float32) * (pb + c0), axis=1, keepdims=True) - 1.0
    p1 = jnp.sum(o1.astype(jnp.float32) * (pb + cnt0 + c1),
                 axis=1, keepdims=True) - 1.0
    p0_ref[...] = p0.astype(jnp.int32)
    p1_ref[...] = p1.astype(jnp.int32)

    jrow = lax.broadcasted_iota(jnp.int32, (128, _E), 0) * _BT
    jrowf = jrow.astype(jnp.float32)
    nonempty = pc > 0
    be = jnp.sum((pb <= jrowf).astype(jnp.int32), axis=1, keepdims=True) - 1
    started = jnp.logical_and(pb <= jrowf, nonempty)
    rix = jnp.sum(started.astype(jnp.int32), axis=1, keepdims=True) - 1
    lanes8 = lax.broadcasted_iota(jnp.int32, (128, _E), 1)
    nxt_cand = jnp.where(jnp.logical_and(pb > jrowf, nonempty), lanes8, 99)
    nxt = jnp.min(nxt_cand, axis=1, keepdims=True)
    nxt = jnp.where(nxt == 99, -1, nxt)
    total = jnp.sum(pc, axis=1, keepdims=True)
    nbu = jnp.broadcast_to(total // _BT, (128, 1))
    lane_id = lax.broadcasted_iota(jnp.int32, (128, 4), 1)
    blk_ref[...] = jnp.where(
        lane_id == 0, be,
        jnp.where(lane_id == 1, nxt, jnp.where(lane_id == 2, rix, nbu)))


def _routing(logits):
    return pl.pallas_call(
        _routing_body,
        out_shape=(
            jax.ShapeDtypeStruct((_N, 1), jnp.int32),
            jax.ShapeDtypeStruct((_N, 1), jnp.int32),
            jax.ShapeDtypeStruct((_N, 1), jnp.float32),
            jax.ShapeDtypeStruct((_N, 1), jnp.float32),
            jax.ShapeDtypeStruct((128, 4), jnp.int32),
        ),
    )(logits)


def _sc_dispatch(xf, ppos):
    mesh = plsc.VectorSubcoreMesh(core_axis_name="c", subcore_axis_name="s")

    @functools.partial(
        pl.kernel,
        out_type=jax.ShapeDtypeStruct((_R, _D), jnp.float32),
        mesh=mesh,
        scratch_types=[
            pltpu.VMEM((_CH,), jnp.int32),
            pltpu.VMEM((_CH, _D), jnp.float32),
            pltpu.SemaphoreType.DMA,
        ],
    )
    def k(x_hbm, idx_hbm, out_hbm, idx_v, rows_v, sem):
        wid = lax.axis_index("s") * 2 + lax.axis_index("c")
        base = wid * _CH
        tok = lax.rem(base, _N)
        pltpu.sync_copy(idx_hbm.at[pl.ds(base, _CH)], idx_v)
        pltpu.async_copy(x_hbm.at[pl.ds(tok, _CH)], rows_v, sem).wait()
        pltpu.sync_copy(rows_v, out_hbm.at[idx_v])

    return k(xf, ppos)


def _sc_collect(y, ppos):
    mesh = plsc.VectorSubcoreMesh(core_axis_name="c", subcore_axis_name="s")

    @functools.partial(
        pl.kernel,
        out_type=jax.ShapeDtypeStruct((_K * _N, _D), jnp.float32),
        mesh=mesh,
        scratch_types=[
            pltpu.VMEM((_CH,), jnp.int32),
            pltpu.VMEM((_CH, _D), jnp.float32),
            pltpu.SemaphoreType.DMA,
        ],
    )
    def k(y_hbm, idx_hbm, out_hbm, idx_v, rows_v, sem):
        wid = lax.axis_index("s") * 2 + lax.axis_index("c")
        base = wid * _CH
        pltpu.sync_copy(idx_hbm.at[pl.ds(base, _CH)], idx_v)
        pltpu.async_copy(y_hbm.at[idx_v], rows_v, sem).wait()
        pltpu.sync_copy(rows_v, out_hbm.at[pl.ds(base, _CH)])

    return k(y, ppos)


def _start_weights(w1_ref, w3_ref, w2_ref, w1f, w3f, w2f, sems, e, slot):
    pltpu.make_async_copy(w1_ref.at[e], w1f.at[slot], sems.at[slot]).start()
    pltpu.make_async_copy(w3_ref.at[e], w3f.at[slot], sems.at[slot]).start()
    pltpu.make_async_copy(w2_ref.at[e], w2f.at[slot], sems.at[slot]).start()


def _wait_weights(w1_ref, w3_ref, w2_ref, w1f, w3f, w2f, sems, e, slot):
    pltpu.make_async_copy(w1_ref.at[e], w1f.at[slot], sems.at[slot]).wait()
    pltpu.make_async_copy(w3_ref.at[e], w3f.at[slot], sems.at[slot]).wait()
    pltpu.make_async_copy(w2_ref.at[e], w2f.at[slot], sems.at[slot]).wait()


def _ffn_body(be_ref, nxt_ref, rix_ref, nbu_ref, xg_ref,
              w1_ref, w3_ref, w2_ref, y_ref,
              w1f, w3f, w2f, sems):
    j = pl.program_id(0)

    @pl.when(j < nbu_ref[0])
    def _():
        cur = be_ref[j]
        slot = lax.rem(rix_ref[j], 2)
        runfirst = jnp.logical_or(j == 0, be_ref[jnp.maximum(j - 1, 0)] != cur)

        @pl.when(j == 0)
        def _():
            _start_weights(w1_ref, w3_ref, w2_ref, w1f, w3f, w2f, sems,
                           cur, slot)

        @pl.when(jnp.logical_and(runfirst, nxt_ref[j] >= 0))
        def _():
            _start_weights(w1_ref, w3_ref, w2_ref, w1f, w3f, w2f, sems,
                           nxt_ref[j], 1 - slot)

        @pl.when(runfirst)
        def _():
            _wait_weights(w1_ref, w3_ref, w2_ref, w1f, w3f, w2f, sems,
                          cur, slot)

        xb = xg_ref[...]
        acc = jnp.zeros((_BT, _D), jnp.float32)
        for f in range(_F // _FC):
            fs = pl.ds(f * _FC, _FC)
            a = jnp.dot(xb, w1f[slot, :, fs],
                        preferred_element_type=jnp.float32)
            b = jnp.dot(xb, w3f[slot, :, fs],
                        preferred_element_type=jnp.float32)
            h = jax.nn.silu(a) * b
            acc = acc + jnp.dot(h, w2f[slot, fs, :],
                                preferred_element_type=jnp.float32)
        y_ref[...] = acc


def _ffn(blkexp, nxtrun, runix, nbused, xg, W1, W3, W2):
    return pl.pallas_call(
        _ffn_body,
        grid_spec=pltpu.PrefetchScalarGridSpec(
            num_scalar_prefetch=4,
            grid=(_NB,),
            in_specs=[
                pl.BlockSpec((_BT, _D), lambda j, *_: (j, 0)),
                pl.BlockSpec(memory_space=pltpu.MemorySpace.HBM),
                pl.BlockSpec(memory_space=pltpu.MemorySpace.HBM),
                pl.BlockSpec(memory_space=pltpu.MemorySpace.HBM),
            ],
            out_specs=pl.BlockSpec((_BT, _D), lambda j, *_: (j, 0)),
            scratch_shapes=[
                pltpu.VMEM((2, _D, _F), jnp.float32),
                pltpu.VMEM((2, _D, _F), jnp.float32),
                pltpu.VMEM((2, _F, _D), jnp.float32),
                pltpu.SemaphoreType.DMA((2,)),
            ],
        ),
        out_shape=jax.ShapeDtypeStruct((_R, _D), jnp.float32),
        compiler_params=pltpu.CompilerParams(
            dimension_semantics=("arbitrary",),
        ),
    )(blkexp, nxtrun, runix, nbused, xg, W1, W3, W2)


_BC = 512


def _combine_body(y0_ref, y1_ref, w0_ref, w1_ref, o_ref):
    o_ref[...] = w0_ref[...] * y0_ref[...] + w1_ref[...] * y1_ref[...]


def _combine(yg, w0, w1):
    nb = _N // _BC
    return pl.pallas_call(
        _combine_body,
        grid=(nb,),
        in_specs=[
            pl.BlockSpec((_BC, _D), lambda t: (t, 0)),
            pl.BlockSpec((_BC, _D), lambda t: (t + nb, 0)),
            pl.BlockSpec((_BC, 1), lambda t: (t, 0)),
            pl.BlockSpec((_BC, 1), lambda t: (t, 0)),
        ],
        out_specs=pl.BlockSpec((_BC, _D), lambda t: (t, 0)),
        out_shape=jax.ShapeDtypeStruct((_N, _D), jnp.float32),
        compiler_params=pltpu.CompilerParams(
            dimension_semantics=("parallel",),
        ),
    )(yg, yg, w0, w1)


def kernel(x, Wg, W1, W3, W2):
    xf = x.reshape(_N, _D)
    logits = xf @ Wg
    p0, p1, w0, w1, blk = _routing(logits)
    ppos = jnp.concatenate([p0, p1], axis=0).reshape(_K * _N)
    blkexp = blk[:_NB, 0]
    nxtrun = blk[:_NB, 1]
    runix = blk[:_NB, 2]
    nbused = blk[0:1, 3].reshape(1)
    xg = _sc_dispatch(xf, ppos)
    y = _ffn(blkexp, nxtrun, runix, nbused, xg, W1, W3, W2)
    yg = _sc_collect(y, ppos)
    out = _combine(yg, w0, w1)
    return out.reshape(x.shape)
